# baseline (speedup 1.0000x reference)
import functools

import jax
import jax.numpy as jnp
from jax import lax
from jax.experimental import pallas as pl
from jax.experimental.pallas import tpu as pltpu

F32 = jnp.float32


def _pick_block(n, target, multiple):
    best = None
    for d in range(1, n + 1):
        if n % d == 0 and d % multiple == 0 and d <= target:
            best = d
    if best is None:
        best = n
    return best


def _pass0_body(params_ref, x1_ref, x2_ref, we1_ref, we2_ref,
                g1a_ref, g1b_ref, g2a_ref, g2b_ref, c1_ref, c2_ref):
    i = pl.program_id(0)
    f1 = jnp.dot(x1_ref[...], we1_ref[...], preferred_element_type=F32)
    f2 = jnp.dot(x2_ref[...], we2_ref[...], preferred_element_type=F32)
    g1a_ref[...] = params_ref[0, 0] * f1
    g1b_ref[...] = params_ref[0, 1] * f1
    g2a_ref[...] = params_ref[0, 3] * f2
    g2b_ref[...] = params_ref[0, 4] * f2

    @pl.when(i == 0)
    def _():
        c1_ref[...] = jnp.zeros_like(c1_ref)
        c2_ref[...] = jnp.zeros_like(c2_ref)

    c1_ref[...] += params_ref[0, 2] * jnp.sum(f1, axis=0, keepdims=True)
    c2_ref[...] += params_ref[0, 5] * jnp.sum(f2, axis=0, keepdims=True)


def _pass1_body(n, bm, bk, as1_ref, af1_ref, as2_ref, af2_ref,
                g1a_ref, g1b_ref, g2a_ref, g2b_ref, c1_ref, c2_ref,
                wm1_ref, bm1_ref, wm2_ref, bm2_ref,
                l1_ref, l2_ref, comb_ref):
    k = pl.program_id(1)
    nk = pl.num_programs(1)
    ks = pl.ds(k * bk, bk)

    def tiles(masked):
        a1, b1 = as1_ref[...], af1_ref[...]
        a2, b2 = as2_ref[...], af2_ref[...]
        if masked:
            valid = lax.broadcasted_iota(jnp.int32, (bm, bk), 1) < (n - k * bk)
            zero = jnp.zeros((), F32)
            a1 = jnp.where(valid, a1, zero)
            b1 = jnp.where(valid, b1, zero)
            a2 = jnp.where(valid, a2, zero)
            b2 = jnp.where(valid, b2, zero)
        p1 = (jnp.dot(a1, g1a_ref[ks, :], preferred_element_type=F32) +
              jnp.dot(b1, g1b_ref[ks, :], preferred_element_type=F32))
        p2 = (jnp.dot(a2, g2a_ref[ks, :], preferred_element_type=F32) +
              jnp.dot(b2, g2b_ref[ks, :], preferred_element_type=F32))
        return p1, p2

    def accumulate(p1, p2):
        @pl.when(k == 0)
        def _():
            l1_ref[...] = c1_ref[...] + p1
            l2_ref[...] = c2_ref[...] + p2

        @pl.when(k != 0)
        def _():
            l1_ref[...] += p1
            l2_ref[...] += p2

    @pl.when(k != nk - 1)
    def _():
        accumulate(*tiles(masked=False))

    @pl.when(k == nk - 1)
    def _():
        accumulate(*tiles(masked=True))
        l1 = l1_ref[...]
        l2 = l2_ref[...]
        o = wm1_ref.shape[1]
        h = (jnp.dot(l1, wm1_ref[0:o, :], preferred_element_type=F32) +
             jnp.dot(l2, wm1_ref[o:2 * o, :], preferred_element_type=F32) +
             bm1_ref[...])
        comb_ref[...] = jnp.dot(h, wm2_ref[...], preferred_element_type=F32) \
            + bm2_ref[...]


def _pass2_body(n, bm, bk, as1_ref, as2_ref, comb_ref, wd1_ref, wd2_ref,
                r1_ref, r2_ref, acc1_ref, acc2_ref):
    k = pl.program_id(1)
    nk = pl.num_programs(1)
    ks = pl.ds(k * bk, bk)

    def tiles(masked):
        a1, a2 = as1_ref[...], as2_ref[...]
        if masked:
            valid = lax.broadcasted_iota(jnp.int32, (bm, bk), 1) < (n - k * bk)
            zero = jnp.zeros((), F32)
            a1 = jnp.where(valid, a1, zero)
            a2 = jnp.where(valid, a2, zero)
        q1 = jnp.dot(a1, comb_ref[ks, :], preferred_element_type=F32)
        q2 = jnp.dot(a2, comb_ref[ks, :], preferred_element_type=F32)
        return q1, q2

    def accumulate(q1, q2):
        @pl.when(k == 0)
        def _():
            acc1_ref[...] = q1
            acc2_ref[...] = q2

        @pl.when(k != 0)
        def _():
            acc1_ref[...] += q1
            acc2_ref[...] += q2

    @pl.when(k != nk - 1)
    def _():
        accumulate(*tiles(masked=False))

    @pl.when(k == nk - 1)
    def _():
        accumulate(*tiles(masked=True))
        r1_ref[...] = jnp.dot(acc1_ref[...], wd1_ref[...],
                              preferred_element_type=F32)
        r2_ref[...] = jnp.dot(acc2_ref[...], wd2_ref[...],
                              preferred_element_type=F32)


def kernel(features_omics1, features_omics2, adj_spatial_omics1,
           adj_feature_omics1, adj_spatial_omics2, adj_feature_omics2,
           cw1, cb1, cw2, cb2, We1, We2, Wd1, Wd2, Wm1, bm1, Wm2, bm2):
    n, d1 = features_omics1.shape
    d2 = features_omics2.shape[1]
    o = We1.shape[1]

    params = jnp.concatenate([
        cw1.astype(F32), jnp.reshape(cb1, (1,)).astype(F32),
        cw2.astype(F32), jnp.reshape(cb2, (1,)).astype(F32),
    ]).reshape(1, 6)

    bm0 = _pick_block(n, 2000, 8)
    nm0 = n // bm0
    g1a, g1b, g2a, g2b, c1, c2 = pl.pallas_call(
        _pass0_body,
        grid=(nm0,),
        in_specs=[
            pl.BlockSpec(memory_space=pltpu.SMEM),
            pl.BlockSpec((bm0, d1), lambda i: (i, 0)),
            pl.BlockSpec((bm0, d2), lambda i: (i, 0)),
            pl.BlockSpec((d1, o), lambda i: (0, 0)),
            pl.BlockSpec((d2, o), lambda i: (0, 0)),
        ],
        out_specs=[
            pl.BlockSpec((bm0, o), lambda i: (i, 0)),
            pl.BlockSpec((bm0, o), lambda i: (i, 0)),
            pl.BlockSpec((bm0, o), lambda i: (i, 0)),
            pl.BlockSpec((bm0, o), lambda i: (i, 0)),
            pl.BlockSpec((1, o), lambda i: (0, 0)),
            pl.BlockSpec((1, o), lambda i: (0, 0)),
        ],
        out_shape=[
            jax.ShapeDtypeStruct((n, o), F32),
            jax.ShapeDtypeStruct((n, o), F32),
            jax.ShapeDtypeStruct((n, o), F32),
            jax.ShapeDtypeStruct((n, o), F32),
            jax.ShapeDtypeStruct((1, o), F32),
            jax.ShapeDtypeStruct((1, o), F32),
        ],
        compiler_params=pltpu.CompilerParams(
            dimension_semantics=("arbitrary",)),
    )(params, features_omics1, features_omics2, We1, We2)

    bm_1 = _pick_block(n, 400, 8)
    bk1 = 512
    nm1, nk1 = n // bm_1, -(-n // bk1)
    kpad1 = nk1 * bk1 - n
    pad = lambda a, p: jnp.pad(a, ((0, p), (0, 0))) if p else a
    adj_spec = pl.BlockSpec((bm_1, bk1), lambda i, k: (i, k))
    thin_spec = pl.BlockSpec((nk1 * bk1, o), lambda i, k: (0, 0))
    row_spec = pl.BlockSpec((1, o), lambda i, k: (0, 0))
    l1, l2, comb = pl.pallas_call(
        functools.partial(_pass1_body, n, bm_1, bk1),
        grid=(nm1, nk1),
        in_specs=[
            adj_spec, adj_spec, adj_spec, adj_spec,
            thin_spec, thin_spec, thin_spec, thin_spec,
            row_spec, row_spec,
            pl.BlockSpec((2 * o, o), lambda i, k: (0, 0)),
            row_spec,
            pl.BlockSpec((o, o), lambda i, k: (0, 0)),
            row_spec,
        ],
        out_specs=[
            pl.BlockSpec((bm_1, o), lambda i, k: (i, 0)),
            pl.BlockSpec((bm_1, o), lambda i, k: (i, 0)),
            pl.BlockSpec((bm_1, o), lambda i, k: (i, 0)),
        ],
        out_shape=[
            jax.ShapeDtypeStruct((n, o), F32),
            jax.ShapeDtypeStruct((n, o), F32),
            jax.ShapeDtypeStruct((n, o), F32),
        ],
        compiler_params=pltpu.CompilerParams(
            dimension_semantics=("parallel", "arbitrary")),
    )(adj_spatial_omics1, adj_feature_omics1, adj_spatial_omics2,
      adj_feature_omics2, pad(g1a, kpad1), pad(g1b, kpad1),
      pad(g2a, kpad1), pad(g2b, kpad1), c1, c2,
      Wm1, bm1.reshape(1, o), Wm2, bm2.reshape(1, o))

    bm_2 = _pick_block(n, 1000, 8)
    bk2 = 512
    nm2, nk2 = n // bm_2, -(-n // bk2)
    kpad2 = nk2 * bk2 - n
    adj2_spec = pl.BlockSpec((bm_2, bk2), lambda i, k: (i, k))
    r1, r2 = pl.pallas_call(
        functools.partial(_pass2_body, n, bm_2, bk2),
        grid=(nm2, nk2),
        in_specs=[
            adj2_spec, adj2_spec,
            pl.BlockSpec((nk2 * bk2, o), lambda i, k: (0, 0)),
            pl.BlockSpec((o, d1), lambda i, k: (0, 0)),
            pl.BlockSpec((o, d2), lambda i, k: (0, 0)),
        ],
        out_specs=[
            pl.BlockSpec((bm_2, d1), lambda i, k: (i, 0)),
            pl.BlockSpec((bm_2, d2), lambda i, k: (i, 0)),
        ],
        out_shape=[
            jax.ShapeDtypeStruct((n, d1), F32),
            jax.ShapeDtypeStruct((n, d2), F32),
        ],
        scratch_shapes=[
            pltpu.VMEM((bm_2, o), F32),
            pltpu.VMEM((bm_2, o), F32),
        ],
        compiler_params=pltpu.CompilerParams(
            dimension_semantics=("parallel", "arbitrary")),
    )(adj_spatial_omics1, adj_spatial_omics2, pad(comb, kpad2), Wd1, Wd2)

    return l1, l2, comb, r1, r2


# speedup vs baseline: 1.0079x; 1.0079x over previous
import functools

import jax
import jax.numpy as jnp
from jax import lax
from jax.experimental import pallas as pl
from jax.experimental.pallas import tpu as pltpu

F32 = jnp.float32
BF16 = jnp.bfloat16


def _pick_block(n, target, multiple):
    best = None
    for d in range(1, n + 1):
        if n % d == 0 and d % multiple == 0 and d <= target:
            best = d
    if best is None:
        best = n
    return best


def _pass0_body(params_ref, x1_ref, x2_ref, we1_ref, we2_ref,
                g1a_ref, g1b_ref, g2a_ref, g2b_ref, c1_ref, c2_ref):
    i = pl.program_id(0)
    f1 = jnp.dot(x1_ref[...], we1_ref[...], preferred_element_type=F32)
    f2 = jnp.dot(x2_ref[...], we2_ref[...], preferred_element_type=F32)
    g1a_ref[...] = (params_ref[0, 0] * f1).astype(BF16)
    g1b_ref[...] = (params_ref[0, 1] * f1).astype(BF16)
    g2a_ref[...] = (params_ref[0, 3] * f2).astype(BF16)
    g2b_ref[...] = (params_ref[0, 4] * f2).astype(BF16)

    @pl.when(i == 0)
    def _():
        c1_ref[...] = jnp.zeros_like(c1_ref)
        c2_ref[...] = jnp.zeros_like(c2_ref)

    c1_ref[...] += params_ref[0, 2] * jnp.sum(f1, axis=0, keepdims=True)
    c2_ref[...] += params_ref[0, 5] * jnp.sum(f2, axis=0, keepdims=True)


def _pass1_body(n, bm, bk, as1_ref, af1_ref, as2_ref, af2_ref,
                g1a_ref, g1b_ref, g2a_ref, g2b_ref, c1_ref, c2_ref,
                wm1_ref, bm1_ref, wm2_ref, bm2_ref,
                l1_ref, l2_ref, comb_ref, combb_ref):
    k = pl.program_id(1)
    nk = pl.num_programs(1)
    ks = pl.ds(k * bk, bk)

    def tiles(masked):
        a1, b1 = as1_ref[...], af1_ref[...]
        a2, b2 = as2_ref[...], af2_ref[...]
        if masked:
            valid = lax.broadcasted_iota(jnp.int32, (bm, bk), 1) < (n - k * bk)
            zero = jnp.zeros((), F32)
            a1 = jnp.where(valid, a1, zero)
            b1 = jnp.where(valid, b1, zero)
            a2 = jnp.where(valid, a2, zero)
            b2 = jnp.where(valid, b2, zero)
        p1 = (jnp.dot(a1.astype(BF16), g1a_ref[ks, :],
                      preferred_element_type=F32) +
              jnp.dot(b1.astype(BF16), g1b_ref[ks, :],
                      preferred_element_type=F32))
        p2 = (jnp.dot(a2.astype(BF16), g2a_ref[ks, :],
                      preferred_element_type=F32) +
              jnp.dot(b2.astype(BF16), g2b_ref[ks, :],
                      preferred_element_type=F32))
        return p1, p2

    def accumulate(p1, p2):
        @pl.when(k == 0)
        def _():
            l1_ref[...] = c1_ref[...] + p1
            l2_ref[...] = c2_ref[...] + p2

        @pl.when(k != 0)
        def _():
            l1_ref[...] += p1
            l2_ref[...] += p2

    @pl.when(k != nk - 1)
    def _():
        accumulate(*tiles(masked=False))

    @pl.when(k == nk - 1)
    def _():
        accumulate(*tiles(masked=True))
        l1 = l1_ref[...]
        l2 = l2_ref[...]
        o = wm1_ref.shape[1]
        h = (jnp.dot(l1, wm1_ref[0:o, :], preferred_element_type=F32) +
             jnp.dot(l2, wm1_ref[o:2 * o, :], preferred_element_type=F32) +
             bm1_ref[...])
        comb = jnp.dot(h, wm2_ref[...], preferred_element_type=F32) \
            + bm2_ref[...]
        comb_ref[...] = comb
        combb_ref[...] = comb.astype(BF16)


def _pass2_body(n, bm, bk, as1_ref, as2_ref, comb_ref, wd1_ref, wd2_ref,
                r1_ref, r2_ref, acc1_ref, acc2_ref):
    k = pl.program_id(1)
    nk = pl.num_programs(1)
    ks = pl.ds(k * bk, bk)

    def tiles(masked):
        a1, a2 = as1_ref[...], as2_ref[...]
        if masked:
            valid = lax.broadcasted_iota(jnp.int32, (bm, bk), 1) < (n - k * bk)
            zero = jnp.zeros((), F32)
            a1 = jnp.where(valid, a1, zero)
            a2 = jnp.where(valid, a2, zero)
        cb = comb_ref[ks, :]
        q1 = jnp.dot(a1.astype(BF16), cb, preferred_element_type=F32)
        q2 = jnp.dot(a2.astype(BF16), cb, preferred_element_type=F32)
        return q1, q2

    def accumulate(q1, q2):
        @pl.when(k == 0)
        def _():
            acc1_ref[...] = q1
            acc2_ref[...] = q2

        @pl.when(k != 0)
        def _():
            acc1_ref[...] += q1
            acc2_ref[...] += q2

    @pl.when(k != nk - 1)
    def _():
        accumulate(*tiles(masked=False))

    @pl.when(k == nk - 1)
    def _():
        accumulate(*tiles(masked=True))
        r1_ref[...] = jnp.dot(acc1_ref[...], wd1_ref[...],
                              preferred_element_type=F32)
        r2_ref[...] = jnp.dot(acc2_ref[...], wd2_ref[...],
                              preferred_element_type=F32)


def kernel(features_omics1, features_omics2, adj_spatial_omics1,
           adj_feature_omics1, adj_spatial_omics2, adj_feature_omics2,
           cw1, cb1, cw2, cb2, We1, We2, Wd1, Wd2, Wm1, bm1, Wm2, bm2):
    n, d1 = features_omics1.shape
    d2 = features_omics2.shape[1]
    o = We1.shape[1]

    params = jnp.concatenate([
        cw1.astype(F32), jnp.reshape(cb1, (1,)).astype(F32),
        cw2.astype(F32), jnp.reshape(cb2, (1,)).astype(F32),
    ]).reshape(1, 6)

    bm0 = _pick_block(n, 2000, 8)
    nm0 = n // bm0
    g1a, g1b, g2a, g2b, c1, c2 = pl.pallas_call(
        _pass0_body,
        grid=(nm0,),
        in_specs=[
            pl.BlockSpec(memory_space=pltpu.SMEM),
            pl.BlockSpec((bm0, d1), lambda i: (i, 0)),
            pl.BlockSpec((bm0, d2), lambda i: (i, 0)),
            pl.BlockSpec((d1, o), lambda i: (0, 0)),
            pl.BlockSpec((d2, o), lambda i: (0, 0)),
        ],
        out_specs=[
            pl.BlockSpec((bm0, o), lambda i: (i, 0)),
            pl.BlockSpec((bm0, o), lambda i: (i, 0)),
            pl.BlockSpec((bm0, o), lambda i: (i, 0)),
            pl.BlockSpec((bm0, o), lambda i: (i, 0)),
            pl.BlockSpec((1, o), lambda i: (0, 0)),
            pl.BlockSpec((1, o), lambda i: (0, 0)),
        ],
        out_shape=[
            jax.ShapeDtypeStruct((n, o), BF16),
            jax.ShapeDtypeStruct((n, o), BF16),
            jax.ShapeDtypeStruct((n, o), BF16),
            jax.ShapeDtypeStruct((n, o), BF16),
            jax.ShapeDtypeStruct((1, o), F32),
            jax.ShapeDtypeStruct((1, o), F32),
        ],
        compiler_params=pltpu.CompilerParams(
            dimension_semantics=("arbitrary",)),
    )(params, features_omics1, features_omics2, We1, We2)

    bm_1 = _pick_block(n, 400, 8)
    bk1 = 512
    nm1, nk1 = n // bm_1, -(-n // bk1)
    kpad1 = nk1 * bk1 - n
    pad = lambda a, p: jnp.pad(a, ((0, p), (0, 0))) if p else a
    adj_spec = pl.BlockSpec((bm_1, bk1), lambda i, k: (i, k))
    thin_spec = pl.BlockSpec((nk1 * bk1, o), lambda i, k: (0, 0))
    row_spec = pl.BlockSpec((1, o), lambda i, k: (0, 0))
    l1, l2, comb, combb = pl.pallas_call(
        functools.partial(_pass1_body, n, bm_1, bk1),
        grid=(nm1, nk1),
        in_specs=[
            adj_spec, adj_spec, adj_spec, adj_spec,
            thin_spec, thin_spec, thin_spec, thin_spec,
            row_spec, row_spec,
            pl.BlockSpec((2 * o, o), lambda i, k: (0, 0)),
            row_spec,
            pl.BlockSpec((o, o), lambda i, k: (0, 0)),
            row_spec,
        ],
        out_specs=[
            pl.BlockSpec((bm_1, o), lambda i, k: (i, 0)),
            pl.BlockSpec((bm_1, o), lambda i, k: (i, 0)),
            pl.BlockSpec((bm_1, o), lambda i, k: (i, 0)),
            pl.BlockSpec((bm_1, o), lambda i, k: (i, 0)),
        ],
        out_shape=[
            jax.ShapeDtypeStruct((n, o), F32),
            jax.ShapeDtypeStruct((n, o), F32),
            jax.ShapeDtypeStruct((n, o), F32),
            jax.ShapeDtypeStruct((n, o), BF16),
        ],
        compiler_params=pltpu.CompilerParams(
            dimension_semantics=("parallel", "arbitrary")),
    )(adj_spatial_omics1, adj_feature_omics1, adj_spatial_omics2,
      adj_feature_omics2, pad(g1a, kpad1), pad(g1b, kpad1),
      pad(g2a, kpad1), pad(g2b, kpad1), c1, c2,
      Wm1, bm1.reshape(1, o), Wm2, bm2.reshape(1, o))

    bm_2 = _pick_block(n, 1000, 8)
    bk2 = 512
    nm2, nk2 = n // bm_2, -(-n // bk2)
    kpad2 = nk2 * bk2 - n
    adj2_spec = pl.BlockSpec((bm_2, bk2), lambda i, k: (i, k))
    r1, r2 = pl.pallas_call(
        functools.partial(_pass2_body, n, bm_2, bk2),
        grid=(nm2, nk2),
        in_specs=[
            adj2_spec, adj2_spec,
            pl.BlockSpec((nk2 * bk2, o), lambda i, k: (0, 0)),
            pl.BlockSpec((o, d1), lambda i, k: (0, 0)),
            pl.BlockSpec((o, d2), lambda i, k: (0, 0)),
        ],
        out_specs=[
            pl.BlockSpec((bm_2, d1), lambda i, k: (i, 0)),
            pl.BlockSpec((bm_2, d2), lambda i, k: (i, 0)),
        ],
        out_shape=[
            jax.ShapeDtypeStruct((n, d1), F32),
            jax.ShapeDtypeStruct((n, d2), F32),
        ],
        scratch_shapes=[
            pltpu.VMEM((bm_2, o), F32),
            pltpu.VMEM((bm_2, o), F32),
        ],
        compiler_params=pltpu.CompilerParams(
            dimension_semantics=("parallel", "arbitrary")),
    )(adj_spatial_omics1, adj_spatial_omics2, pad(combb, kpad2), Wd1, Wd2)

    return l1, l2, comb, r1, r2


# speedup vs baseline: 1.1927x; 1.1834x over previous
import functools

import jax
import jax.numpy as jnp
from jax import lax
from jax.experimental import pallas as pl
from jax.experimental.pallas import tpu as pltpu

F32 = jnp.float32
BF16 = jnp.bfloat16


def _pick_block(n, target, multiple):
    best = None
    for d in range(1, n + 1):
        if n % d == 0 and d % multiple == 0 and d <= target:
            best = d
    if best is None:
        best = n
    return best


def _pass0_body(params_ref, x1_ref, x2_ref, we1_ref, we2_ref,
                g1a_ref, g1b_ref, g2a_ref, g2b_ref, c1_ref, c2_ref):
    i = pl.program_id(0)
    f1 = jnp.dot(x1_ref[...], we1_ref[...], preferred_element_type=F32)
    f2 = jnp.dot(x2_ref[...], we2_ref[...], preferred_element_type=F32)
    g1a_ref[...] = params_ref[0, 0] * f1
    g1b_ref[...] = params_ref[0, 1] * f1
    g2a_ref[...] = params_ref[0, 3] * f2
    g2b_ref[...] = params_ref[0, 4] * f2

    @pl.when(i == 0)
    def _():
        c1_ref[...] = jnp.zeros_like(c1_ref)
        c2_ref[...] = jnp.zeros_like(c2_ref)

    c1_ref[...] += params_ref[0, 2] * jnp.sum(f1, axis=0, keepdims=True)
    c2_ref[...] += params_ref[0, 5] * jnp.sum(f2, axis=0, keepdims=True)


def _pass1_body(n, bm, bk, as1_ref, af1_ref, as2_ref, af2_ref,
                g1_ref, g2_ref, c1_ref, c2_ref,
                wm1_ref, bm1_ref, wm2_ref, bm2_ref,
                l1_ref, l2_ref, comb_ref, combb_ref):
    k = pl.program_id(1)
    nk = pl.num_programs(1)
    ks = pl.ds(k * 2 * bk, 2 * bk)

    def tiles(masked):
        a1, b1 = as1_ref[...], af1_ref[...]
        a2, b2 = as2_ref[...], af2_ref[...]
        if masked:
            valid = lax.broadcasted_iota(jnp.int32, (bm, bk), 1) < (n - k * bk)
            zero = jnp.zeros((), F32)
            a1 = jnp.where(valid, a1, zero)
            b1 = jnp.where(valid, b1, zero)
            a2 = jnp.where(valid, a2, zero)
            b2 = jnp.where(valid, b2, zero)
        kh = pl.ds(k * 2 * bk, bk)
        kl = pl.ds(k * 2 * bk + bk, bk)
        p1 = (jnp.dot(a1, g1_ref[kh, :], preferred_element_type=F32,
                      precision=lax.Precision.DEFAULT) +
              jnp.dot(b1, g1_ref[kl, :], preferred_element_type=F32,
                      precision=lax.Precision.DEFAULT))
        p2 = (jnp.dot(a2, g2_ref[kh, :], preferred_element_type=F32,
                      precision=lax.Precision.DEFAULT) +
              jnp.dot(b2, g2_ref[kl, :], preferred_element_type=F32,
                      precision=lax.Precision.DEFAULT))
        return p1, p2

    def accumulate(p1, p2):
        @pl.when(k == 0)
        def _():
            l1_ref[...] = c1_ref[...] + p1
            l2_ref[...] = c2_ref[...] + p2

        @pl.when(k != 0)
        def _():
            l1_ref[...] += p1
            l2_ref[...] += p2

    @pl.when(k != nk - 1)
    def _():
        accumulate(*tiles(masked=False))

    @pl.when(k == nk - 1)
    def _():
        accumulate(*tiles(masked=True))
        l1 = l1_ref[...]
        l2 = l2_ref[...]
        o = wm1_ref.shape[1]
        h = (jnp.dot(l1, wm1_ref[0:o, :], preferred_element_type=F32) +
             jnp.dot(l2, wm1_ref[o:2 * o, :], preferred_element_type=F32) +
             bm1_ref[...])
        comb = jnp.dot(h, wm2_ref[...], preferred_element_type=F32) \
            + bm2_ref[...]
        comb_ref[...] = comb
        combb_ref[...] = comb


def _pass2_body(n, bm, bk, as1_ref, as2_ref, comb_ref, wd1_ref, wd2_ref,
                r1_ref, r2_ref, acc1_ref, acc2_ref):
    k = pl.program_id(1)
    nk = pl.num_programs(1)
    ks = pl.ds(k * bk, bk)

    def tiles(masked):
        a1, a2 = as1_ref[...], as2_ref[...]
        if masked:
            valid = lax.broadcasted_iota(jnp.int32, (bm, bk), 1) < (n - k * bk)
            zero = jnp.zeros((), F32)
            a1 = jnp.where(valid, a1, zero)
            a2 = jnp.where(valid, a2, zero)
        cb = comb_ref[ks, :]
        q1 = jnp.dot(a1, cb, preferred_element_type=F32,
                     precision=lax.Precision.DEFAULT)
        q2 = jnp.dot(a2, cb, preferred_element_type=F32,
                     precision=lax.Precision.DEFAULT)
        return q1, q2

    def accumulate(q1, q2):
        @pl.when(k == 0)
        def _():
            acc1_ref[...] = q1
            acc2_ref[...] = q2

        @pl.when(k != 0)
        def _():
            acc1_ref[...] += q1
            acc2_ref[...] += q2

    @pl.when(k != nk - 1)
    def _():
        accumulate(*tiles(masked=False))

    @pl.when(k == nk - 1)
    def _():
        accumulate(*tiles(masked=True))
        r1_ref[...] = jnp.dot(acc1_ref[...], wd1_ref[...],
                              preferred_element_type=F32)
        r2_ref[...] = jnp.dot(acc2_ref[...], wd2_ref[...],
                              preferred_element_type=F32)


def kernel(features_omics1, features_omics2, adj_spatial_omics1,
           adj_feature_omics1, adj_spatial_omics2, adj_feature_omics2,
           cw1, cb1, cw2, cb2, We1, We2, Wd1, Wd2, Wm1, bm1, Wm2, bm2):
    n, d1 = features_omics1.shape
    d2 = features_omics2.shape[1]
    o = We1.shape[1]

    params = jnp.concatenate([
        cw1.astype(F32), jnp.reshape(cb1, (1,)).astype(F32),
        cw2.astype(F32), jnp.reshape(cb2, (1,)).astype(F32),
    ]).reshape(1, 6)

    bm0 = _pick_block(n, 2000, 8)
    nm0 = n // bm0
    g1a, g1b, g2a, g2b, c1, c2 = pl.pallas_call(
        _pass0_body,
        grid=(nm0,),
        in_specs=[
            pl.BlockSpec(memory_space=pltpu.SMEM),
            pl.BlockSpec((bm0, d1), lambda i: (i, 0)),
            pl.BlockSpec((bm0, d2), lambda i: (i, 0)),
            pl.BlockSpec((d1, o), lambda i: (0, 0)),
            pl.BlockSpec((d2, o), lambda i: (0, 0)),
        ],
        out_specs=[
            pl.BlockSpec((bm0, o), lambda i: (i, 0)),
            pl.BlockSpec((bm0, o), lambda i: (i, 0)),
            pl.BlockSpec((bm0, o), lambda i: (i, 0)),
            pl.BlockSpec((bm0, o), lambda i: (i, 0)),
            pl.BlockSpec((1, o), lambda i: (0, 0)),
            pl.BlockSpec((1, o), lambda i: (0, 0)),
        ],
        out_shape=[
            jax.ShapeDtypeStruct((n, o), F32),
            jax.ShapeDtypeStruct((n, o), F32),
            jax.ShapeDtypeStruct((n, o), F32),
            jax.ShapeDtypeStruct((n, o), F32),
            jax.ShapeDtypeStruct((1, o), F32),
            jax.ShapeDtypeStruct((1, o), F32),
        ],
        compiler_params=pltpu.CompilerParams(
            dimension_semantics=("arbitrary",)),
    )(params, features_omics1, features_omics2, We1, We2)

    bm_1 = _pick_block(n, 1000, 8)
    bk1 = 512
    nm1, nk1 = n // bm_1, -(-n // bk1)
    kpad1 = nk1 * bk1 - n
    pad = lambda a, p: jnp.pad(a, ((0, p), (0, 0))) if p else a

    def interleave(ga, gb):
        ga = pad(ga, kpad1).reshape(nk1, bk1, o)
        gb = pad(gb, kpad1).reshape(nk1, bk1, o)
        return jnp.concatenate([ga, gb], axis=1).reshape(2 * nk1 * bk1, o)

    g1 = interleave(g1a, g1b)
    g2 = interleave(g2a, g2b)
    adj_spec = pl.BlockSpec((bm_1, bk1), lambda i, k: (i, k))
    thin_spec = pl.BlockSpec((2 * nk1 * bk1, o), lambda i, k: (0, 0))
    row_spec = pl.BlockSpec((1, o), lambda i, k: (0, 0))
    l1, l2, comb, combb = pl.pallas_call(
        functools.partial(_pass1_body, n, bm_1, bk1),
        grid=(nm1, nk1),
        in_specs=[
            adj_spec, adj_spec, adj_spec, adj_spec,
            thin_spec, thin_spec,
            row_spec, row_spec,
            pl.BlockSpec((2 * o, o), lambda i, k: (0, 0)),
            row_spec,
            pl.BlockSpec((o, o), lambda i, k: (0, 0)),
            row_spec,
        ],
        out_specs=[
            pl.BlockSpec((bm_1, o), lambda i, k: (i, 0)),
            pl.BlockSpec((bm_1, o), lambda i, k: (i, 0)),
            pl.BlockSpec((bm_1, o), lambda i, k: (i, 0)),
            pl.BlockSpec((bm_1, o), lambda i, k: (i, 0)),
        ],
        out_shape=[
            jax.ShapeDtypeStruct((n, o), F32),
            jax.ShapeDtypeStruct((n, o), F32),
            jax.ShapeDtypeStruct((n, o), F32),
            jax.ShapeDtypeStruct((n, o), F32),
        ],
        compiler_params=pltpu.CompilerParams(
            dimension_semantics=("parallel", "arbitrary")),
    )(adj_spatial_omics1, adj_feature_omics1, adj_spatial_omics2,
      adj_feature_omics2, g1, g2, c1, c2,
      Wm1, bm1.reshape(1, o), Wm2, bm2.reshape(1, o))

    bm_2 = _pick_block(n, 1000, 8)
    bk2 = 512
    nm2, nk2 = n // bm_2, -(-n // bk2)
    kpad2 = nk2 * bk2 - n
    adj2_spec = pl.BlockSpec((bm_2, bk2), lambda i, k: (i, k))
    r1, r2 = pl.pallas_call(
        functools.partial(_pass2_body, n, bm_2, bk2),
        grid=(nm2, nk2),
        in_specs=[
            adj2_spec, adj2_spec,
            pl.BlockSpec((nk2 * bk2, o), lambda i, k: (0, 0)),
            pl.BlockSpec((o, d1), lambda i, k: (0, 0)),
            pl.BlockSpec((o, d2), lambda i, k: (0, 0)),
        ],
        out_specs=[
            pl.BlockSpec((bm_2, d1), lambda i, k: (i, 0)),
            pl.BlockSpec((bm_2, d2), lambda i, k: (i, 0)),
        ],
        out_shape=[
            jax.ShapeDtypeStruct((n, d1), F32),
            jax.ShapeDtypeStruct((n, d2), F32),
        ],
        scratch_shapes=[
            pltpu.VMEM((bm_2, o), F32),
            pltpu.VMEM((bm_2, o), F32),
        ],
        compiler_params=pltpu.CompilerParams(
            dimension_semantics=("parallel", "arbitrary")),
    )(adj_spatial_omics1, adj_spatial_omics2, pad(combb, kpad2), Wd1, Wd2)

    return l1, l2, comb, r1, r2


# speedup vs baseline: 1.2990x; 1.0891x over previous
import functools

import jax
import jax.numpy as jnp
from jax import lax
from jax.experimental import pallas as pl
from jax.experimental.pallas import tpu as pltpu

F32 = jnp.float32

_DN = (((1,), (1,)), ((), ()))


def _dott(thin, big):
    return lax.dot_general(thin, big, dimension_numbers=_DN,
                           preferred_element_type=F32,
                           precision=lax.Precision.DEFAULT)


def _pass0_body(n, bm0, params_ref, x1_ref, x2_ref, we1_ref, we2_ref,
                g1a_ref, g1b_ref, g2a_ref, g2b_ref, c1_ref, c2_ref):
    i = pl.program_id(0)
    f1 = lax.dot_general(we1_ref[...], x1_ref[...],
                         dimension_numbers=(((0,), (1,)), ((), ())),
                         preferred_element_type=F32)
    f2 = lax.dot_general(we2_ref[...], x2_ref[...],
                         dimension_numbers=(((0,), (1,)), ((), ())),
                         preferred_element_type=F32)
    g1a_ref[...] = params_ref[0, 0] * f1
    g1b_ref[...] = params_ref[0, 1] * f1
    g2a_ref[...] = params_ref[0, 3] * f2
    g2b_ref[...] = params_ref[0, 4] * f2

    @pl.when(i == 0)
    def _():
        c1_ref[...] = jnp.zeros_like(c1_ref)
        c2_ref[...] = jnp.zeros_like(c2_ref)

    o = f1.shape[0]
    valid = lax.broadcasted_iota(jnp.int32, (o, bm0), 1) < (n - i * bm0)
    zero = jnp.zeros((), F32)
    c1_ref[...] += params_ref[0, 2] * jnp.sum(
        jnp.where(valid, f1, zero), axis=1, keepdims=True)
    c2_ref[...] += params_ref[0, 5] * jnp.sum(
        jnp.where(valid, f2, zero), axis=1, keepdims=True)


def _pass1_body(n, bm, bk, as1_ref, af1_ref, as2_ref, af2_ref,
                g1a_ref, g1b_ref, g2a_ref, g2b_ref, c1_ref, c2_ref,
                wm1at_ref, wm1bt_ref, bm1_ref, wm2t_ref, bm2_ref,
                l1_ref, l2_ref, comb_ref):
    k = pl.program_id(1)
    nk = pl.num_programs(1)
    ks = pl.ds(k * bk, bk)

    def tiles(masked):
        a1, b1 = as1_ref[...], af1_ref[...]
        a2, b2 = as2_ref[...], af2_ref[...]
        if masked:
            valid = lax.broadcasted_iota(jnp.int32, (bm, bk), 1) < (n - k * bk)
            zero = jnp.zeros((), F32)
            a1 = jnp.where(valid, a1, zero)
            b1 = jnp.where(valid, b1, zero)
            a2 = jnp.where(valid, a2, zero)
            b2 = jnp.where(valid, b2, zero)
        p1 = _dott(g1a_ref[:, ks], a1) + _dott(g1b_ref[:, ks], b1)
        p2 = _dott(g2a_ref[:, ks], a2) + _dott(g2b_ref[:, ks], b2)
        return p1, p2

    def accumulate(p1, p2):
        @pl.when(k == 0)
        def _():
            l1_ref[...] = c1_ref[...] + p1
            l2_ref[...] = c2_ref[...] + p2

        @pl.when(k != 0)
        def _():
            l1_ref[...] += p1
            l2_ref[...] += p2

    @pl.when(k != nk - 1)
    def _():
        accumulate(*tiles(masked=False))

    @pl.when(k == nk - 1)
    def _():
        accumulate(*tiles(masked=True))
        l1 = l1_ref[...]
        l2 = l2_ref[...]
        h = (jnp.dot(wm1at_ref[...], l1, preferred_element_type=F32) +
             jnp.dot(wm1bt_ref[...], l2, preferred_element_type=F32) +
             bm1_ref[...])
        comb_ref[...] = jnp.dot(wm2t_ref[...], h,
                                preferred_element_type=F32) + bm2_ref[...]


def _pass2_body(n, bm, bk, as1_ref, as2_ref, comb_ref, wd1t_ref, wd2t_ref,
                r1_ref, r2_ref, acc1_ref, acc2_ref):
    k = pl.program_id(1)
    nk = pl.num_programs(1)
    ks = pl.ds(k * bk, bk)

    def tiles(masked):
        a1, a2 = as1_ref[...], as2_ref[...]
        if masked:
            valid = lax.broadcasted_iota(jnp.int32, (bm, bk), 1) < (n - k * bk)
            zero = jnp.zeros((), F32)
            a1 = jnp.where(valid, a1, zero)
            a2 = jnp.where(valid, a2, zero)
        cb = comb_ref[:, ks]
        return _dott(cb, a1), _dott(cb, a2)

    def accumulate(q1, q2):
        @pl.when(k == 0)
        def _():
            acc1_ref[...] = q1
            acc2_ref[...] = q2

        @pl.when(k != 0)
        def _():
            acc1_ref[...] += q1
            acc2_ref[...] += q2

    @pl.when(k != nk - 1)
    def _():
        accumulate(*tiles(masked=False))

    @pl.when(k == nk - 1)
    def _():
        accumulate(*tiles(masked=True))
        r1_ref[...] = jnp.dot(wd1t_ref[...], acc1_ref[...],
                              preferred_element_type=F32)
        r2_ref[...] = jnp.dot(wd2t_ref[...], acc2_ref[...],
                              preferred_element_type=F32)


def kernel(features_omics1, features_omics2, adj_spatial_omics1,
           adj_feature_omics1, adj_spatial_omics2, adj_feature_omics2,
           cw1, cb1, cw2, cb2, We1, We2, Wd1, Wd2, Wm1, bm1, Wm2, bm2):
    n, d1 = features_omics1.shape
    d2 = features_omics2.shape[1]
    o = We1.shape[1]

    params = jnp.concatenate([
        cw1.astype(F32), jnp.reshape(cb1, (1,)).astype(F32),
        cw2.astype(F32), jnp.reshape(cb2, (1,)).astype(F32),
    ]).reshape(1, 6)

    bm0 = 2048
    nm0 = -(-n // bm0)
    g1a, g1b, g2a, g2b, c1, c2 = pl.pallas_call(
        functools.partial(_pass0_body, n, bm0),
        grid=(nm0,),
        in_specs=[
            pl.BlockSpec(memory_space=pltpu.SMEM),
            pl.BlockSpec((bm0, d1), lambda i: (i, 0)),
            pl.BlockSpec((bm0, d2), lambda i: (i, 0)),
            pl.BlockSpec((d1, o), lambda i: (0, 0)),
            pl.BlockSpec((d2, o), lambda i: (0, 0)),
        ],
        out_specs=[
            pl.BlockSpec((o, bm0), lambda i: (0, i)),
            pl.BlockSpec((o, bm0), lambda i: (0, i)),
            pl.BlockSpec((o, bm0), lambda i: (0, i)),
            pl.BlockSpec((o, bm0), lambda i: (0, i)),
            pl.BlockSpec((o, 1), lambda i: (0, 0)),
            pl.BlockSpec((o, 1), lambda i: (0, 0)),
        ],
        out_shape=[
            jax.ShapeDtypeStruct((o, n), F32),
            jax.ShapeDtypeStruct((o, n), F32),
            jax.ShapeDtypeStruct((o, n), F32),
            jax.ShapeDtypeStruct((o, n), F32),
            jax.ShapeDtypeStruct((o, 1), F32),
            jax.ShapeDtypeStruct((o, 1), F32),
        ],
        compiler_params=pltpu.CompilerParams(
            dimension_semantics=("arbitrary",)),
    )(params, features_omics1, features_omics2, We1, We2)

    bm_1 = 1024
    bk1 = 512
    nm1, nk1 = -(-n // bm_1), -(-n // bk1)
    kpad1 = nk1 * bk1 - n
    padc = lambda a, p: jnp.pad(a, ((0, 0), (0, p))) if p else a
    adj_spec = pl.BlockSpec((bm_1, bk1), lambda i, k: (i, k))
    thin_spec = pl.BlockSpec((o, nk1 * bk1), lambda i, k: (0, 0))
    col_spec = pl.BlockSpec((o, 1), lambda i, k: (0, 0))
    sq_spec = pl.BlockSpec((o, o), lambda i, k: (0, 0))
    out1_spec = pl.BlockSpec((o, bm_1), lambda i, k: (0, i))
    l1, l2, comb = pl.pallas_call(
        functools.partial(_pass1_body, n, bm_1, bk1),
        grid=(nm1, nk1),
        in_specs=[
            adj_spec, adj_spec, adj_spec, adj_spec,
            thin_spec, thin_spec, thin_spec, thin_spec,
            col_spec, col_spec,
            sq_spec, sq_spec, col_spec, sq_spec, col_spec,
        ],
        out_specs=[out1_spec, out1_spec, out1_spec],
        out_shape=[
            jax.ShapeDtypeStruct((o, n), F32),
            jax.ShapeDtypeStruct((o, n), F32),
            jax.ShapeDtypeStruct((o, n), F32),
        ],
        compiler_params=pltpu.CompilerParams(
            dimension_semantics=("parallel", "arbitrary")),
    )(adj_spatial_omics1, adj_feature_omics1, adj_spatial_omics2,
      adj_feature_omics2, padc(g1a, kpad1), padc(g1b, kpad1),
      padc(g2a, kpad1), padc(g2b, kpad1), c1, c2,
      Wm1[:o].T, Wm1[o:].T, bm1.reshape(o, 1), Wm2.T, bm2.reshape(o, 1))

    bm_2 = 2048
    bk2 = 512
    nm2, nk2 = -(-n // bm_2), -(-n // bk2)
    kpad2 = nk2 * bk2 - n
    adj2_spec = pl.BlockSpec((bm_2, bk2), lambda i, k: (i, k))
    r1, r2 = pl.pallas_call(
        functools.partial(_pass2_body, n, bm_2, bk2),
        grid=(nm2, nk2),
        in_specs=[
            adj2_spec, adj2_spec,
            pl.BlockSpec((o, nk2 * bk2), lambda i, k: (0, 0)),
            pl.BlockSpec((d1, o), lambda i, k: (0, 0)),
            pl.BlockSpec((d2, o), lambda i, k: (0, 0)),
        ],
        out_specs=[
            pl.BlockSpec((d1, bm_2), lambda i, k: (0, i)),
            pl.BlockSpec((d2, bm_2), lambda i, k: (0, i)),
        ],
        out_shape=[
            jax.ShapeDtypeStruct((d1, n), F32),
            jax.ShapeDtypeStruct((d2, n), F32),
        ],
        scratch_shapes=[
            pltpu.VMEM((o, bm_2), F32),
            pltpu.VMEM((o, bm_2), F32),
        ],
        compiler_params=pltpu.CompilerParams(
            dimension_semantics=("parallel", "arbitrary")),
    )(adj_spatial_omics1, adj_spatial_omics2, padc(comb, kpad2),
      Wd1.T, Wd2.T)

    return l1.T, l2.T, comb.T, r1.T, r2.T


# speedup vs baseline: 1.4360x; 1.1054x over previous
import functools

import jax
import jax.numpy as jnp
from jax import lax
from jax.experimental import pallas as pl
from jax.experimental.pallas import tpu as pltpu

F32 = jnp.float32
BF16 = jnp.bfloat16

_DN = (((1,), (1,)), ((), ()))


def _dott(thin, big):
    return lax.dot_general(thin, big.astype(BF16), dimension_numbers=_DN,
                           preferred_element_type=F32,
                           precision=lax.Precision.DEFAULT)


def _pass0_body(n, bm0, params_ref, x1_ref, x2_ref, we1_ref, we2_ref,
                g1a_ref, g1b_ref, g2a_ref, g2b_ref, c1_ref, c2_ref):
    i = pl.program_id(0)
    f1 = lax.dot_general(we1_ref[...], x1_ref[...],
                         dimension_numbers=(((0,), (1,)), ((), ())),
                         preferred_element_type=F32)
    f2 = lax.dot_general(we2_ref[...], x2_ref[...],
                         dimension_numbers=(((0,), (1,)), ((), ())),
                         preferred_element_type=F32)
    g1a_ref[...] = (params_ref[0, 0] * f1).astype(BF16)
    g1b_ref[...] = (params_ref[0, 1] * f1).astype(BF16)
    g2a_ref[...] = (params_ref[0, 3] * f2).astype(BF16)
    g2b_ref[...] = (params_ref[0, 4] * f2).astype(BF16)

    @pl.when(i == 0)
    def _():
        c1_ref[...] = jnp.zeros_like(c1_ref)
        c2_ref[...] = jnp.zeros_like(c2_ref)

    o = f1.shape[0]
    valid = lax.broadcasted_iota(jnp.int32, (o, bm0), 1) < (n - i * bm0)
    zero = jnp.zeros((), F32)
    c1_ref[...] += params_ref[0, 2] * jnp.sum(
        jnp.where(valid, f1, zero), axis=1, keepdims=True)
    c2_ref[...] += params_ref[0, 5] * jnp.sum(
        jnp.where(valid, f2, zero), axis=1, keepdims=True)


def _pass1_body(n, bm, bk, as1_ref, af1_ref, as2_ref, af2_ref,
                g1a_ref, g1b_ref, g2a_ref, g2b_ref, c1_ref, c2_ref,
                wm1at_ref, wm1bt_ref, bm1_ref, wm2t_ref, bm2_ref,
                l1_ref, l2_ref, comb_ref, combb_ref):
    k = pl.program_id(1)
    nk = pl.num_programs(1)
    ks = pl.ds(k * bk, bk)

    def tiles(masked):
        a1, b1 = as1_ref[...], af1_ref[...]
        a2, b2 = as2_ref[...], af2_ref[...]
        if masked:
            valid = lax.broadcasted_iota(jnp.int32, (bm, bk), 1) < (n - k * bk)
            zero = jnp.zeros((), F32)
            a1 = jnp.where(valid, a1, zero)
            b1 = jnp.where(valid, b1, zero)
            a2 = jnp.where(valid, a2, zero)
            b2 = jnp.where(valid, b2, zero)
        p1 = _dott(g1a_ref[:, ks], a1) + _dott(g1b_ref[:, ks], b1)
        p2 = _dott(g2a_ref[:, ks], a2) + _dott(g2b_ref[:, ks], b2)
        return p1, p2

    def accumulate(p1, p2):
        @pl.when(k == 0)
        def _():
            l1_ref[...] = c1_ref[...] + p1
            l2_ref[...] = c2_ref[...] + p2

        @pl.when(k != 0)
        def _():
            l1_ref[...] += p1
            l2_ref[...] += p2

    @pl.when(k != nk - 1)
    def _():
        accumulate(*tiles(masked=False))

    @pl.when(k == nk - 1)
    def _():
        accumulate(*tiles(masked=True))
        l1 = l1_ref[...]
        l2 = l2_ref[...]
        h = (jnp.dot(wm1at_ref[...], l1, preferred_element_type=F32) +
             jnp.dot(wm1bt_ref[...], l2, preferred_element_type=F32) +
             bm1_ref[...])
        comb = jnp.dot(wm2t_ref[...], h,
                       preferred_element_type=F32) + bm2_ref[...]
        comb_ref[...] = comb
        combb_ref[...] = comb.astype(BF16)


def _pass2_body(n, bm, bk, as1_ref, as2_ref, comb_ref, wd1t_ref, wd2t_ref,
                r1_ref, r2_ref, acc1_ref, acc2_ref):
    k = pl.program_id(1)
    nk = pl.num_programs(1)
    ks = pl.ds(k * bk, bk)

    def tiles(masked):
        a1, a2 = as1_ref[...], as2_ref[...]
        if masked:
            valid = lax.broadcasted_iota(jnp.int32, (bm, bk), 1) < (n - k * bk)
            zero = jnp.zeros((), F32)
            a1 = jnp.where(valid, a1, zero)
            a2 = jnp.where(valid, a2, zero)
        cb = comb_ref[:, ks]
        return _dott(cb, a1), _dott(cb, a2)

    def accumulate(q1, q2):
        @pl.when(k == 0)
        def _():
            acc1_ref[...] = q1
            acc2_ref[...] = q2

        @pl.when(k != 0)
        def _():
            acc1_ref[...] += q1
            acc2_ref[...] += q2

    @pl.when(k != nk - 1)
    def _():
        accumulate(*tiles(masked=False))

    @pl.when(k == nk - 1)
    def _():
        accumulate(*tiles(masked=True))
        r1_ref[...] = jnp.dot(wd1t_ref[...], acc1_ref[...],
                              preferred_element_type=F32)
        r2_ref[...] = jnp.dot(wd2t_ref[...], acc2_ref[...],
                              preferred_element_type=F32)


def kernel(features_omics1, features_omics2, adj_spatial_omics1,
           adj_feature_omics1, adj_spatial_omics2, adj_feature_omics2,
           cw1, cb1, cw2, cb2, We1, We2, Wd1, Wd2, Wm1, bm1, Wm2, bm2):
    n, d1 = features_omics1.shape
    d2 = features_omics2.shape[1]
    o = We1.shape[1]

    params = jnp.concatenate([
        cw1.astype(F32), jnp.reshape(cb1, (1,)).astype(F32),
        cw2.astype(F32), jnp.reshape(cb2, (1,)).astype(F32),
    ]).reshape(1, 6)

    bm0 = 2048
    nm0 = -(-n // bm0)
    g1a, g1b, g2a, g2b, c1, c2 = pl.pallas_call(
        functools.partial(_pass0_body, n, bm0),
        grid=(nm0,),
        in_specs=[
            pl.BlockSpec(memory_space=pltpu.SMEM),
            pl.BlockSpec((bm0, d1), lambda i: (i, 0)),
            pl.BlockSpec((bm0, d2), lambda i: (i, 0)),
            pl.BlockSpec((d1, o), lambda i: (0, 0)),
            pl.BlockSpec((d2, o), lambda i: (0, 0)),
        ],
        out_specs=[
            pl.BlockSpec((o, bm0), lambda i: (0, i)),
            pl.BlockSpec((o, bm0), lambda i: (0, i)),
            pl.BlockSpec((o, bm0), lambda i: (0, i)),
            pl.BlockSpec((o, bm0), lambda i: (0, i)),
            pl.BlockSpec((o, 1), lambda i: (0, 0)),
            pl.BlockSpec((o, 1), lambda i: (0, 0)),
        ],
        out_shape=[
            jax.ShapeDtypeStruct((o, n), BF16),
            jax.ShapeDtypeStruct((o, n), BF16),
            jax.ShapeDtypeStruct((o, n), BF16),
            jax.ShapeDtypeStruct((o, n), BF16),
            jax.ShapeDtypeStruct((o, 1), F32),
            jax.ShapeDtypeStruct((o, 1), F32),
        ],
        compiler_params=pltpu.CompilerParams(
            dimension_semantics=("arbitrary",)),
    )(params, features_omics1, features_omics2, We1, We2)

    bm_1 = 1024
    bk1 = 1024
    nm1, nk1 = -(-n // bm_1), -(-n // bk1)
    kpad1 = nk1 * bk1 - n
    padc = lambda a, p: jnp.pad(a, ((0, 0), (0, p))) if p else a
    adj_spec = pl.BlockSpec((bm_1, bk1), lambda i, k: (i, k))
    thin_spec = pl.BlockSpec((o, nk1 * bk1), lambda i, k: (0, 0))
    col_spec = pl.BlockSpec((o, 1), lambda i, k: (0, 0))
    sq_spec = pl.BlockSpec((o, o), lambda i, k: (0, 0))
    out1_spec = pl.BlockSpec((o, bm_1), lambda i, k: (0, i))
    l1, l2, comb, combb = pl.pallas_call(
        functools.partial(_pass1_body, n, bm_1, bk1),
        grid=(nm1, nk1),
        in_specs=[
            adj_spec, adj_spec, adj_spec, adj_spec,
            thin_spec, thin_spec, thin_spec, thin_spec,
            col_spec, col_spec,
            sq_spec, sq_spec, col_spec, sq_spec, col_spec,
        ],
        out_specs=[out1_spec, out1_spec, out1_spec, out1_spec],
        out_shape=[
            jax.ShapeDtypeStruct((o, n), F32),
            jax.ShapeDtypeStruct((o, n), F32),
            jax.ShapeDtypeStruct((o, n), F32),
            jax.ShapeDtypeStruct((o, n), BF16),
        ],
        compiler_params=pltpu.CompilerParams(
            dimension_semantics=("parallel", "arbitrary")),
    )(adj_spatial_omics1, adj_feature_omics1, adj_spatial_omics2,
      adj_feature_omics2, padc(g1a, kpad1), padc(g1b, kpad1),
      padc(g2a, kpad1), padc(g2b, kpad1), c1, c2,
      Wm1[:o].T, Wm1[o:].T, bm1.reshape(o, 1), Wm2.T, bm2.reshape(o, 1))

    bm_2 = 2048
    bk2 = 1024
    nm2, nk2 = -(-n // bm_2), -(-n // bk2)
    kpad2 = nk2 * bk2 - n
    adj2_spec = pl.BlockSpec((bm_2, bk2), lambda i, k: (i, k))
    r1, r2 = pl.pallas_call(
        functools.partial(_pass2_body, n, bm_2, bk2),
        grid=(nm2, nk2),
        in_specs=[
            adj2_spec, adj2_spec,
            pl.BlockSpec((o, nk2 * bk2), lambda i, k: (0, 0)),
            pl.BlockSpec((d1, o), lambda i, k: (0, 0)),
            pl.BlockSpec((d2, o), lambda i, k: (0, 0)),
        ],
        out_specs=[
            pl.BlockSpec((d1, bm_2), lambda i, k: (0, i)),
            pl.BlockSpec((d2, bm_2), lambda i, k: (0, i)),
        ],
        out_shape=[
            jax.ShapeDtypeStruct((d1, n), F32),
            jax.ShapeDtypeStruct((d2, n), F32),
        ],
        scratch_shapes=[
            pltpu.VMEM((o, bm_2), F32),
            pltpu.VMEM((o, bm_2), F32),
        ],
        compiler_params=pltpu.CompilerParams(
            dimension_semantics=("parallel", "arbitrary")),
    )(adj_spatial_omics1, adj_spatial_omics2, padc(combb, kpad2),
      Wd1.T, Wd2.T)

    return l1.T, l2.T, comb.T, r1.T, r2.T
